# Initial kernel scaffold; baseline (speedup 1.0000x reference)
import jax
import jax.numpy as jnp
from jax.experimental import pallas as pl


def kernel(dense_w, dense_b, w_slab, vec_slab, as_features, bert_last_hidden, bert_pooler, aa_len, map_AA, map_AA_idx, map_AS, map_AS_idx, aa_graph_length, aa_graph):
    raise NotImplementedError("write your pallas kernel here")


# speedup vs baseline: 1.6093x; 1.6093x over previous
import numpy as np
import jax
import jax.numpy as jnp
from jax.experimental import pallas as pl
from jax.experimental.pallas import tpu as pltpu

BERT_HIDDEN = 32
HIDDEN = 16
IN_DIM = BERT_HIDDEN + HIDDEN
NHEAD = 4
DHEAD = IN_DIM // NHEAD
DIM_FF = HIDDEN
LN_EPS = 1e-5


def _node_kernel(x_ref, len_ref, pool_ref, w_ref, b_ref, out_ref):
    TN, S1, _ = x_ref.shape
    S = S1 - 1
    lens = len_ref[...]
    pos = jax.lax.broadcasted_iota(jnp.int32, (TN, S1), 1)
    valid = ((pos >= 1) & (pos <= lens)).astype(jnp.float32)[:, :, None]
    xm_sum = jnp.sum(x_ref[...] * valid, axis=1)
    node = jnp.dot(xm_sum, w_ref[...],
                   preferred_element_type=jnp.float32) + jnp.float32(S) * b_ref[...]
    out_ref[...] = jnp.concatenate([node, pool_ref[...]], axis=-1)


def _node_features(bert_last_hidden, aa_len, pooler, w, b, *, tile_n=128):
    n, S1, Hb = bert_last_hidden.shape
    n_pad = ((n + tile_n - 1) // tile_n) * tile_n
    lens = aa_len.reshape(-1, 1).astype(jnp.int32)
    if n_pad != n:
        p = n_pad - n
        bert_last_hidden = jnp.pad(bert_last_hidden, ((0, p), (0, 0), (0, 0)))
        lens = jnp.pad(lens, ((0, p), (0, 0)))
        pooler = jnp.pad(pooler, ((0, p), (0, 0)))
    out = pl.pallas_call(
        _node_kernel,
        out_shape=jax.ShapeDtypeStruct((n_pad, HIDDEN + BERT_HIDDEN), jnp.float32),
        grid_spec=pltpu.PrefetchScalarGridSpec(
            num_scalar_prefetch=0,
            grid=(n_pad // tile_n,),
            in_specs=[
                pl.BlockSpec((tile_n, S1, Hb), lambda i: (i, 0, 0)),
                pl.BlockSpec((tile_n, 1), lambda i: (i, 0)),
                pl.BlockSpec((tile_n, Hb), lambda i: (i, 0)),
                pl.BlockSpec((Hb, HIDDEN), lambda i: (0, 0)),
                pl.BlockSpec((1, HIDDEN), lambda i: (0, 0)),
            ],
            out_specs=pl.BlockSpec((tile_n, HIDDEN + BERT_HIDDEN), lambda i: (i, 0)),
        ),
        compiler_params=pltpu.CompilerParams(dimension_semantics=("parallel",)),
    )(bert_last_hidden, lens, pooler, w, b)
    return out[:n]


def _encoder_kernel(len_ref, x_ref, g_ref, w_ref, vec_ref, out_ref):
    b = pl.program_id(0)
    x = x_ref[0]
    g = g_ref[0]
    L = x.shape[0]
    D = IN_DIM

    w_qkv = w_ref[0:D, :]
    w_o = w_ref[D:2 * D, 0:D]
    w_ff1 = w_ref[2 * D:3 * D, 0:DIM_FF]
    w_ff2 = w_ref[3 * D:3 * D + DIM_FF, 0:D]
    b_qkv = vec_ref[0:1, :]
    b_o = vec_ref[1:2, 0:D]
    b_ff1 = vec_ref[2:3, 0:DIM_FF]
    b_ff2 = vec_ref[3:4, 0:D]
    ln1_g = vec_ref[4:5, 0:D]
    ln1_b = vec_ref[5:6, 0:D]
    ln2_g = vec_ref[6:7, 0:D]
    ln2_b = vec_ref[7:8, 0:D]

    qkv = jnp.dot(x, w_qkv, preferred_element_type=jnp.float32) + b_qkv
    scale = jnp.float32(1.0 / np.sqrt(DHEAD))
    q = qkv[:, 0:D] * scale
    k = qkv[:, D:2 * D]
    v = qkv[:, 2 * D:3 * D]

    qt = jnp.concatenate([q, q, q, q], axis=0)
    rh = jax.lax.broadcasted_iota(jnp.int32, (NHEAD * L, D), 0) // L
    ch = jax.lax.broadcasted_iota(jnp.int32, (NHEAD * L, D), 1) // DHEAD
    qs = jnp.where(rh == ch, qt, 0.0)
    raw = jax.lax.dot_general(qs, k, (((1,), (1,)), ((), ())),
                              preferred_element_type=jnp.float32)
    e = jnp.exp(raw - jnp.max(raw, axis=1, keepdims=True))

    length = len_ref[b]
    col = jax.lax.broadcasted_iota(jnp.int32, (L, L), 1)
    cv = (col < length).astype(jnp.float32)
    a = (g != 0.0).astype(jnp.float32)
    m_f = a * cv
    m_b = a.T * cv
    ef = e * jnp.concatenate([m_f, m_f, m_f, m_f], axis=0)
    eb = e * jnp.concatenate([m_b, m_b, m_b, m_b], axis=0)

    rawq = raw - jnp.float32(1e9)
    e2 = (rawq >= jnp.max(rawq, axis=1, keepdims=True)).astype(jnp.float32)
    empty_f = jnp.sum(m_f, axis=1, keepdims=True) == 0.0
    empty_b = jnp.sum(m_b, axis=1, keepdims=True) == 0.0
    ef4 = jnp.concatenate([empty_f, empty_f, empty_f, empty_f], axis=0)
    eb4 = jnp.concatenate([empty_b, empty_b, empty_b, empty_b], axis=0)
    ef = jnp.where(ef4, e2, ef)
    eb = jnp.where(eb4, e2, eb)

    es = jnp.concatenate([ef, eb], axis=0)
    vd = jnp.concatenate([v, jnp.ones((L, 1), jnp.float32)], axis=1)
    ctx = jnp.dot(es, vd, preferred_element_type=jnp.float32)
    r = ctx[:, 0:D] / ctx[:, D:D + 1]

    hm = jax.lax.broadcasted_iota(jnp.int32, (L, D), 1) // DHEAD
    ctx_f = jnp.zeros((L, D), jnp.float32)
    ctx_b = jnp.zeros((L, D), jnp.float32)
    for h in range(NHEAD):
        sel = (hm == h).astype(jnp.float32)
        ctx_f = ctx_f + r[h * L:(h + 1) * L] * sel
        ctx_b = ctx_b + r[(NHEAD + h) * L:(NHEAD + h + 1) * L] * sel

    ctx2 = jnp.concatenate([ctx_f, ctx_b], axis=0)
    attn2 = jnp.dot(ctx2, w_o, preferred_element_type=jnp.float32) + b_o

    def layer_norm(y, gg, bb):
        mu = jnp.mean(y, axis=-1, keepdims=True)
        var = jnp.mean((y - mu) ** 2, axis=-1, keepdims=True)
        return (y - mu) * jax.lax.rsqrt(var + LN_EPS) * gg + bb

    x2 = jnp.concatenate([x, x], axis=0)
    x1 = layer_norm(x2 + attn2, ln1_g, ln1_b)
    hidden = jax.nn.relu(
        jnp.dot(x1, w_ff1, preferred_element_type=jnp.float32) + b_ff1)
    ff = jnp.dot(hidden, w_ff2, preferred_element_type=jnp.float32) + b_ff2
    y2 = layer_norm(x1 + ff, ln2_g, ln2_b)

    out_ref[0] = y2[0:L, :] + y2[L:2 * L, :]


def _graph_encoder(graph_in, aa_graph, aa_graph_length, w_slab, vec_slab):
    B, L, D = graph_in.shape
    return pl.pallas_call(
        _encoder_kernel,
        out_shape=jax.ShapeDtypeStruct((B, L, D), jnp.float32),
        grid_spec=pltpu.PrefetchScalarGridSpec(
            num_scalar_prefetch=1,
            grid=(B,),
            in_specs=[
                pl.BlockSpec((1, L, D), lambda b, ln: (b, 0, 0)),
                pl.BlockSpec((1, L, L), lambda b, ln: (b, 0, 0)),
                pl.BlockSpec(w_slab.shape, lambda b, ln: (0, 0)),
                pl.BlockSpec(vec_slab.shape, lambda b, ln: (0, 0)),
            ],
            out_specs=pl.BlockSpec((1, L, D), lambda b, ln: (b, 0, 0)),
        ),
        compiler_params=pltpu.CompilerParams(dimension_semantics=("parallel",)),
    )(aa_graph_length.astype(jnp.int32), graph_in,
      aa_graph.astype(jnp.float32), w_slab, vec_slab)


def kernel(dense_w, dense_b, w_slab, vec_slab, as_features, bert_last_hidden,
           bert_pooler, aa_len, map_AA, map_AA_idx, map_AS, map_AS_idx,
           aa_graph_length, aa_graph):
    B, L, _ = aa_graph.shape

    rela_v_inner = _node_features(bert_last_hidden, aa_len, bert_pooler,
                                  dense_w, dense_b)

    graph_in = jnp.zeros((B, L, IN_DIM), jnp.float32)
    graph_in = graph_in.at[map_AA, map_AA_idx].set(rela_v_inner)
    graph_in = graph_in.at[map_AS, map_AS_idx].set(as_features)

    mutual = _graph_encoder(graph_in, aa_graph, aa_graph_length,
                            w_slab, vec_slab)

    AA_features = mutual[map_AS, map_AS_idx]
    need = (aa_graph_length[map_AS] > 1).astype(jnp.float32)[:, None]
    blended = AA_features * need + as_features * (1.0 - need)
    return blended + as_features


# speedup vs baseline: 1.6584x; 1.0305x over previous
import numpy as np
import jax
import jax.numpy as jnp
from jax.experimental import pallas as pl
from jax.experimental.pallas import tpu as pltpu

BERT_HIDDEN = 32
HIDDEN = 16
IN_DIM = BERT_HIDDEN + HIDDEN
NHEAD = 4
DHEAD = IN_DIM // NHEAD
DIM_FF = HIDDEN
LN_EPS = 1e-5


def _node_kernel(x_ref, len_ref, pool_ref, w_ref, b_ref, out_ref):
    TN, S1, _ = x_ref.shape
    S = S1 - 1
    lens = len_ref[...]
    pos = jax.lax.broadcasted_iota(jnp.int32, (TN, S1), 1)
    valid = ((pos >= 1) & (pos <= lens)).astype(jnp.float32)[:, :, None]
    xm_sum = jnp.sum(x_ref[...] * valid, axis=1)
    node = jnp.dot(xm_sum, w_ref[...],
                   preferred_element_type=jnp.float32) + jnp.float32(S) * b_ref[...]
    out_ref[...] = jnp.concatenate([node, pool_ref[...]], axis=-1)


def _node_features(bert_last_hidden, aa_len, pooler, w, b, *, tile_n=128):
    n, S1, Hb = bert_last_hidden.shape
    n_pad = ((n + tile_n - 1) // tile_n) * tile_n
    lens = aa_len.reshape(-1, 1).astype(jnp.int32)
    if n_pad != n:
        p = n_pad - n
        bert_last_hidden = jnp.pad(bert_last_hidden, ((0, p), (0, 0), (0, 0)))
        lens = jnp.pad(lens, ((0, p), (0, 0)))
        pooler = jnp.pad(pooler, ((0, p), (0, 0)))
    out = pl.pallas_call(
        _node_kernel,
        out_shape=jax.ShapeDtypeStruct((n_pad, HIDDEN + BERT_HIDDEN), jnp.float32),
        grid_spec=pltpu.PrefetchScalarGridSpec(
            num_scalar_prefetch=0,
            grid=(n_pad // tile_n,),
            in_specs=[
                pl.BlockSpec((tile_n, S1, Hb), lambda i: (i, 0, 0)),
                pl.BlockSpec((tile_n, 1), lambda i: (i, 0)),
                pl.BlockSpec((tile_n, Hb), lambda i: (i, 0)),
                pl.BlockSpec((Hb, HIDDEN), lambda i: (0, 0)),
                pl.BlockSpec((1, HIDDEN), lambda i: (0, 0)),
            ],
            out_specs=pl.BlockSpec((tile_n, HIDDEN + BERT_HIDDEN), lambda i: (i, 0)),
        ),
        compiler_params=pltpu.CompilerParams(dimension_semantics=("parallel",)),
    )(bert_last_hidden, lens, pooler, w, b)
    return out[:n]


def _encoder_kernel(len_ref, x_ref, g_ref, w_ref, vec_ref, out_ref):
    b = pl.program_id(0)
    x = x_ref[0]
    g = g_ref[0]
    L = x.shape[0]
    D = IN_DIM

    w_qkv = w_ref[0:D, :]
    w_o = w_ref[D:2 * D, 0:D]
    w_ff1 = w_ref[2 * D:3 * D, 0:DIM_FF]
    w_ff2 = w_ref[3 * D:3 * D + DIM_FF, 0:D]
    b_qkv = vec_ref[0:1, :]
    b_o = vec_ref[1:2, 0:D]
    b_ff1 = vec_ref[2:3, 0:DIM_FF]
    b_ff2 = vec_ref[3:4, 0:D]
    ln1_g = vec_ref[4:5, 0:D]
    ln1_b = vec_ref[5:6, 0:D]
    ln2_g = vec_ref[6:7, 0:D]
    ln2_b = vec_ref[7:8, 0:D]

    qkv = jnp.dot(x, w_qkv, preferred_element_type=jnp.float32) + b_qkv
    scale = jnp.float32(1.0 / np.sqrt(DHEAD))
    q = qkv[:, 0:D] * scale
    k = qkv[:, D:2 * D]
    v = qkv[:, 2 * D:3 * D]

    kt = jnp.concatenate([k, k, k, k], axis=0)
    rh = jax.lax.broadcasted_iota(jnp.int32, (NHEAD * L, D), 0) // L
    ch = jax.lax.broadcasted_iota(jnp.int32, (NHEAD * L, D), 1) // DHEAD
    kst = jnp.where(rh == ch, kt, 0.0)
    rawl = jax.lax.dot_general(q, kst, (((1,), (1,)), ((), ())),
                               preferred_element_type=jnp.float32)

    length = len_ref[b]
    col = jax.lax.broadcasted_iota(jnp.int32, (L, L), 1)
    cv = (col < length).astype(jnp.float32)
    a = (g != 0.0).astype(jnp.float32)
    m_f = a * cv
    m_b = a.T * cv
    empty_f = jnp.sum(m_f, axis=1, keepdims=True) == 0.0
    empty_b = jnp.sum(m_b, axis=1, keepdims=True) == 0.0

    efs = []
    ebs = []
    big = jnp.float32(1e9)
    for h in range(NHEAD):
        raw_h = rawl[:, h * L:(h + 1) * L]
        mx = jnp.max(raw_h, axis=1, keepdims=True)
        e = jnp.exp(raw_h - mx)
        e2 = ((raw_h - big) >= (mx - big)).astype(jnp.float32)
        efs.append(jnp.where(empty_f, e2, e * m_f).astype(jnp.bfloat16))
        ebs.append(jnp.where(empty_b, e2, e * m_b).astype(jnp.bfloat16))
    ef = jnp.concatenate(efs, axis=1)
    eb = jnp.concatenate(ebs, axis=1)

    vt = jnp.concatenate([v, v, v, v], axis=0)
    vst = jnp.where(rh == ch, vt, 0.0)
    r4 = jax.lax.broadcasted_iota(jnp.int32, (NHEAD * L, NHEAD), 0) // L
    c4 = jax.lax.broadcasted_iota(jnp.int32, (NHEAD * L, NHEAD), 1)
    den_ind = (r4 == c4).astype(jnp.float32)
    vd = jnp.concatenate([vst, den_ind], axis=1).astype(jnp.bfloat16)

    ctf = jnp.dot(ef, vd, preferred_element_type=jnp.float32)
    ctb = jnp.dot(eb, vd, preferred_element_type=jnp.float32)
    outs = []
    for ct in (ctf, ctb):
        parts = [ct[:, h * DHEAD:(h + 1) * DHEAD] / ct[:, D + h:D + h + 1]
                 for h in range(NHEAD)]
        outs.append(jnp.concatenate(parts, axis=1))
    ctx2 = jnp.concatenate(outs, axis=0)
    attn2 = jnp.dot(ctx2, w_o, preferred_element_type=jnp.float32) + b_o

    def layer_norm(y, gg, bb):
        mu = jnp.mean(y, axis=-1, keepdims=True)
        var = jnp.mean((y - mu) ** 2, axis=-1, keepdims=True)
        return (y - mu) * jax.lax.rsqrt(var + LN_EPS) * gg + bb

    x2 = jnp.concatenate([x, x], axis=0)
    x1 = layer_norm(x2 + attn2, ln1_g, ln1_b)
    hidden = jax.nn.relu(
        jnp.dot(x1, w_ff1, preferred_element_type=jnp.float32) + b_ff1)
    ff = jnp.dot(hidden, w_ff2, preferred_element_type=jnp.float32) + b_ff2
    y2 = layer_norm(x1 + ff, ln2_g, ln2_b)

    out_ref[0] = y2[0:L, :] + y2[L:2 * L, :]


def _graph_encoder(graph_in, aa_graph, aa_graph_length, w_slab, vec_slab):
    B, L, D = graph_in.shape
    return pl.pallas_call(
        _encoder_kernel,
        out_shape=jax.ShapeDtypeStruct((B, L, D), jnp.float32),
        grid_spec=pltpu.PrefetchScalarGridSpec(
            num_scalar_prefetch=1,
            grid=(B,),
            in_specs=[
                pl.BlockSpec((1, L, D), lambda b, ln: (b, 0, 0)),
                pl.BlockSpec((1, L, L), lambda b, ln: (b, 0, 0)),
                pl.BlockSpec(w_slab.shape, lambda b, ln: (0, 0)),
                pl.BlockSpec(vec_slab.shape, lambda b, ln: (0, 0)),
            ],
            out_specs=pl.BlockSpec((1, L, D), lambda b, ln: (b, 0, 0)),
        ),
        compiler_params=pltpu.CompilerParams(dimension_semantics=("parallel",)),
    )(aa_graph_length.astype(jnp.int32), graph_in,
      aa_graph.astype(jnp.float32), w_slab, vec_slab)


def kernel(dense_w, dense_b, w_slab, vec_slab, as_features, bert_last_hidden,
           bert_pooler, aa_len, map_AA, map_AA_idx, map_AS, map_AS_idx,
           aa_graph_length, aa_graph):
    B, L, _ = aa_graph.shape

    rela_v_inner = _node_features(bert_last_hidden, aa_len, bert_pooler,
                                  dense_w, dense_b)

    graph_in = jnp.zeros((B, L, IN_DIM), jnp.float32)
    graph_in = graph_in.at[map_AA, map_AA_idx].set(rela_v_inner)
    graph_in = graph_in.at[map_AS, map_AS_idx].set(as_features)

    mutual = _graph_encoder(graph_in, aa_graph, aa_graph_length,
                            w_slab, vec_slab)

    AA_features = mutual[map_AS, map_AS_idx]
    need = (aa_graph_length[map_AS] > 1).astype(jnp.float32)[:, None]
    blended = AA_features * need + as_features * (1.0 - need)
    return blended + as_features


# speedup vs baseline: 1.7620x; 1.0625x over previous
import numpy as np
import jax
import jax.numpy as jnp
from jax.experimental import pallas as pl
from jax.experimental.pallas import tpu as pltpu

BERT_HIDDEN = 32
HIDDEN = 16
IN_DIM = BERT_HIDDEN + HIDDEN
NHEAD = 4
DHEAD = IN_DIM // NHEAD
DIM_FF = HIDDEN
LN_EPS = 1e-5


def _node_kernel(x_ref, len_ref, pool_ref, w_ref, b_ref, out_ref):
    TN, S1, _ = x_ref.shape
    S = S1 - 1
    lens = len_ref[...]
    pos = jax.lax.broadcasted_iota(jnp.int32, (TN, S1), 1)
    valid = ((pos >= 1) & (pos <= lens)).astype(jnp.float32)[:, :, None]
    xm_sum = jnp.sum(x_ref[...] * valid, axis=1)
    node = jnp.dot(xm_sum, w_ref[...],
                   preferred_element_type=jnp.float32) + jnp.float32(S) * b_ref[...]
    out_ref[...] = jnp.concatenate([node, pool_ref[...]], axis=-1)


def _node_features(bert_last_hidden, aa_len, pooler, w, b, *, tile_n=128):
    n, S1, Hb = bert_last_hidden.shape
    n_pad = ((n + tile_n - 1) // tile_n) * tile_n
    lens = aa_len.reshape(-1, 1).astype(jnp.int32)
    if n_pad != n:
        p = n_pad - n
        bert_last_hidden = jnp.pad(bert_last_hidden, ((0, p), (0, 0), (0, 0)))
        lens = jnp.pad(lens, ((0, p), (0, 0)))
        pooler = jnp.pad(pooler, ((0, p), (0, 0)))
    out = pl.pallas_call(
        _node_kernel,
        out_shape=jax.ShapeDtypeStruct((n_pad, HIDDEN + BERT_HIDDEN), jnp.float32),
        grid_spec=pltpu.PrefetchScalarGridSpec(
            num_scalar_prefetch=0,
            grid=(n_pad // tile_n,),
            in_specs=[
                pl.BlockSpec((tile_n, S1, Hb), lambda i: (i, 0, 0)),
                pl.BlockSpec((tile_n, 1), lambda i: (i, 0)),
                pl.BlockSpec((tile_n, Hb), lambda i: (i, 0)),
                pl.BlockSpec((Hb, HIDDEN), lambda i: (0, 0)),
                pl.BlockSpec((1, HIDDEN), lambda i: (0, 0)),
            ],
            out_specs=pl.BlockSpec((tile_n, HIDDEN + BERT_HIDDEN), lambda i: (i, 0)),
        ),
        compiler_params=pltpu.CompilerParams(dimension_semantics=("parallel",)),
    )(bert_last_hidden, lens, pooler, w, b)
    return out[:n]


def _encoder_kernel(len_ref, x_ref, g_ref, w_ref, vec_ref, out_ref):
    b = pl.program_id(0)
    x = x_ref[0]
    g = g_ref[0]
    L = x.shape[0]
    D = IN_DIM

    w_qkv = w_ref[0:D, :]
    w_o = w_ref[D:2 * D, 0:D]
    w_ff1 = w_ref[2 * D:3 * D, 0:DIM_FF]
    w_ff2 = w_ref[3 * D:3 * D + DIM_FF, 0:D]
    b_qkv = vec_ref[0:1, :]
    b_o = vec_ref[1:2, 0:D]
    b_ff1 = vec_ref[2:3, 0:DIM_FF]
    b_ff2 = vec_ref[3:4, 0:D]
    ln1_g = vec_ref[4:5, 0:D]
    ln1_b = vec_ref[5:6, 0:D]
    ln2_g = vec_ref[6:7, 0:D]
    ln2_b = vec_ref[7:8, 0:D]

    qkv = jnp.dot(x, w_qkv, preferred_element_type=jnp.float32) + b_qkv
    scale = jnp.float32(1.0 / np.sqrt(DHEAD))
    q = qkv[:, 0:D] * scale
    k = qkv[:, D:2 * D]
    v = qkv[:, 2 * D:3 * D]

    kt = jnp.concatenate([k, k, k, k], axis=0)
    rh = jax.lax.broadcasted_iota(jnp.int32, (NHEAD * L, D), 0) // L
    ch = jax.lax.broadcasted_iota(jnp.int32, (NHEAD * L, D), 1) // DHEAD
    kst = jnp.where(rh == ch, kt, 0.0)
    rawl = jax.lax.dot_general(q, kst, (((1,), (1,)), ((), ())),
                               preferred_element_type=jnp.float32)

    length = len_ref[b]
    col = jax.lax.broadcasted_iota(jnp.int32, (L, L), 1)
    cv = (col < length).astype(jnp.float32)
    a = (g != 0.0).astype(jnp.float32)
    m_f = a * cv
    m_b = a.T * cv
    empty_f = jnp.sum(m_f, axis=1, keepdims=True) == 0.0
    empty_b = jnp.sum(m_b, axis=1, keepdims=True) == 0.0

    efs = []
    ebs = []
    big = jnp.float32(1e9)
    for h in range(NHEAD):
        raw_h = rawl[:, h * L:(h + 1) * L]
        mx = jnp.max(raw_h, axis=1, keepdims=True)
        e = jnp.exp(raw_h - mx)
        e2 = ((raw_h - big) >= (mx - big)).astype(jnp.float32)
        efs.append(jnp.where(empty_f, e2, e * m_f).astype(jnp.bfloat16))
        ebs.append(jnp.where(empty_b, e2, e * m_b).astype(jnp.bfloat16))
    ef = jnp.concatenate(efs, axis=1)
    eb = jnp.concatenate(ebs, axis=1)

    vt = jnp.concatenate([v, v, v, v], axis=0)
    vst = jnp.where(rh == ch, vt, 0.0)
    r4 = jax.lax.broadcasted_iota(jnp.int32, (NHEAD * L, NHEAD), 0) // L
    c4 = jax.lax.broadcasted_iota(jnp.int32, (NHEAD * L, NHEAD), 1)
    den_ind = (r4 == c4).astype(jnp.float32)
    vd = jnp.concatenate([vst, den_ind], axis=1).astype(jnp.bfloat16)

    ctf = jnp.dot(ef, vd, preferred_element_type=jnp.float32)
    ctb = jnp.dot(eb, vd, preferred_element_type=jnp.float32)
    outs = []
    for ct in (ctf, ctb):
        rec = 1.0 / ct[:, D:D + NHEAD]
        parts = [ct[:, h * DHEAD:(h + 1) * DHEAD] * rec[:, h:h + 1]
                 for h in range(NHEAD)]
        outs.append(jnp.concatenate(parts, axis=1))
    ctx2 = jnp.concatenate(outs, axis=0)
    attn2 = jnp.dot(ctx2, w_o, preferred_element_type=jnp.float32) + b_o

    def layer_norm(y, gg, bb):
        mu = jnp.mean(y, axis=-1, keepdims=True)
        m2 = jnp.mean(y * y, axis=-1, keepdims=True)
        var = m2 - mu * mu
        return (y - mu) * jax.lax.rsqrt(var + LN_EPS) * gg + bb

    x2 = jnp.concatenate([x, x], axis=0)
    x1 = layer_norm(x2 + attn2, ln1_g, ln1_b)
    hidden = jax.nn.relu(
        jnp.dot(x1, w_ff1, preferred_element_type=jnp.float32) + b_ff1)
    ff = jnp.dot(hidden, w_ff2, preferred_element_type=jnp.float32) + b_ff2
    y2 = layer_norm(x1 + ff, ln2_g, ln2_b)

    out_ref[0] = y2[0:L, :] + y2[L:2 * L, :]


def _graph_encoder(graph_in, aa_graph, aa_graph_length, w_slab, vec_slab):
    B, L, D = graph_in.shape
    return pl.pallas_call(
        _encoder_kernel,
        out_shape=jax.ShapeDtypeStruct((B, L, D), jnp.float32),
        grid_spec=pltpu.PrefetchScalarGridSpec(
            num_scalar_prefetch=1,
            grid=(B,),
            in_specs=[
                pl.BlockSpec((1, L, D), lambda b, ln: (b, 0, 0)),
                pl.BlockSpec((1, L, L), lambda b, ln: (b, 0, 0)),
                pl.BlockSpec(w_slab.shape, lambda b, ln: (0, 0)),
                pl.BlockSpec(vec_slab.shape, lambda b, ln: (0, 0)),
            ],
            out_specs=pl.BlockSpec((1, L, D), lambda b, ln: (b, 0, 0)),
        ),
        compiler_params=pltpu.CompilerParams(dimension_semantics=("parallel",)),
    )(aa_graph_length.astype(jnp.int32), graph_in,
      aa_graph.astype(jnp.float32), w_slab, vec_slab)


def kernel(dense_w, dense_b, w_slab, vec_slab, as_features, bert_last_hidden,
           bert_pooler, aa_len, map_AA, map_AA_idx, map_AS, map_AS_idx,
           aa_graph_length, aa_graph):
    B, L, _ = aa_graph.shape

    rela_v_inner = _node_features(bert_last_hidden, aa_len, bert_pooler,
                                  dense_w, dense_b)

    graph_in = jnp.zeros((B, L, IN_DIM), jnp.float32)
    graph_in = graph_in.at[map_AA, map_AA_idx].set(rela_v_inner)
    graph_in = graph_in.at[map_AS, map_AS_idx].set(as_features)

    mutual = _graph_encoder(graph_in, aa_graph, aa_graph_length,
                            w_slab, vec_slab)

    AA_features = mutual[map_AS, map_AS_idx]
    need = (aa_graph_length[map_AS] > 1).astype(jnp.float32)[:, None]
    blended = AA_features * need + as_features * (1.0 - need)
    return blended + as_features


# speedup vs baseline: 1.8800x; 1.0670x over previous
import numpy as np
import jax
import jax.numpy as jnp
from jax.experimental import pallas as pl
from jax.experimental.pallas import tpu as pltpu

BERT_HIDDEN = 32
HIDDEN = 16
IN_DIM = BERT_HIDDEN + HIDDEN
NHEAD = 4
DHEAD = IN_DIM // NHEAD
DIM_FF = HIDDEN
LN_EPS = 1e-5


def _node_kernel(x_ref, len_ref, pool_ref, w_ref, b_ref, out_ref):
    TN, S1, _ = x_ref.shape
    S = S1 - 1
    lens = len_ref[...]
    pos = jax.lax.broadcasted_iota(jnp.int32, (TN, S1), 1)
    valid = ((pos >= 1) & (pos <= lens)).astype(jnp.float32)[:, :, None]
    xm_sum = jnp.sum(x_ref[...] * valid, axis=1)
    node = jnp.dot(xm_sum, w_ref[...],
                   preferred_element_type=jnp.float32) + jnp.float32(S) * b_ref[...]
    out_ref[...] = jnp.concatenate([node, pool_ref[...]], axis=-1)


def _node_features(bert_last_hidden, aa_len, pooler, w, b, *, tile_n=128):
    n, S1, Hb = bert_last_hidden.shape
    n_pad = ((n + tile_n - 1) // tile_n) * tile_n
    lens = aa_len.reshape(-1, 1).astype(jnp.int32)
    if n_pad != n:
        p = n_pad - n
        bert_last_hidden = jnp.pad(bert_last_hidden, ((0, p), (0, 0), (0, 0)))
        lens = jnp.pad(lens, ((0, p), (0, 0)))
        pooler = jnp.pad(pooler, ((0, p), (0, 0)))
    out = pl.pallas_call(
        _node_kernel,
        out_shape=jax.ShapeDtypeStruct((n_pad, HIDDEN + BERT_HIDDEN), jnp.float32),
        grid_spec=pltpu.PrefetchScalarGridSpec(
            num_scalar_prefetch=0,
            grid=(n_pad // tile_n,),
            in_specs=[
                pl.BlockSpec((tile_n, S1, Hb), lambda i: (i, 0, 0)),
                pl.BlockSpec((tile_n, 1), lambda i: (i, 0)),
                pl.BlockSpec((tile_n, Hb), lambda i: (i, 0)),
                pl.BlockSpec((Hb, HIDDEN), lambda i: (0, 0)),
                pl.BlockSpec((1, HIDDEN), lambda i: (0, 0)),
            ],
            out_specs=pl.BlockSpec((tile_n, HIDDEN + BERT_HIDDEN), lambda i: (i, 0)),
        ),
        compiler_params=pltpu.CompilerParams(dimension_semantics=("parallel",)),
    )(bert_last_hidden, lens, pooler, w, b)
    return out[:n]


def _encoder_kernel(len_ref, x_ref, g_ref, w_ref, vec_ref, out_ref):
    nb = x_ref.shape[0]
    for i in range(nb):
        _encode_one(len_ref, x_ref, g_ref, w_ref, vec_ref, out_ref,
                    pl.program_id(0) * nb + i, i)


def _encode_one(len_ref, x_ref, g_ref, w_ref, vec_ref, out_ref, b, i):
    x = x_ref[i]
    g = g_ref[i]
    L = x.shape[0]
    D = IN_DIM

    w_qkv = w_ref[0:D, :]
    w_o = w_ref[D:2 * D, 0:D]
    w_ff1 = w_ref[2 * D:3 * D, 0:DIM_FF]
    w_ff2 = w_ref[3 * D:3 * D + DIM_FF, 0:D]
    b_qkv = vec_ref[0:1, :]
    b_o = vec_ref[1:2, 0:D]
    b_ff1 = vec_ref[2:3, 0:DIM_FF]
    b_ff2 = vec_ref[3:4, 0:D]
    ln1_g = vec_ref[4:5, 0:D]
    ln1_b = vec_ref[5:6, 0:D]
    ln2_g = vec_ref[6:7, 0:D]
    ln2_b = vec_ref[7:8, 0:D]

    qkv = jnp.dot(x, w_qkv, preferred_element_type=jnp.float32) + b_qkv
    scale = jnp.float32(1.0 / np.sqrt(DHEAD))
    q = qkv[:, 0:D] * scale
    k = qkv[:, D:2 * D]
    v = qkv[:, 2 * D:3 * D]

    kt = jnp.concatenate([k, k, k, k], axis=0)
    rh = jax.lax.broadcasted_iota(jnp.int32, (NHEAD * L, D), 0) // L
    ch = jax.lax.broadcasted_iota(jnp.int32, (NHEAD * L, D), 1) // DHEAD
    kst = jnp.where(rh == ch, kt, 0.0)
    rawl = jax.lax.dot_general(q, kst, (((1,), (1,)), ((), ())),
                               preferred_element_type=jnp.float32)

    length = len_ref[b]
    col = jax.lax.broadcasted_iota(jnp.int32, (L, L), 1)
    cv = (col < length).astype(jnp.float32)
    a = (g != 0.0).astype(jnp.float32)
    m_f = a * cv
    m_b = a.T * cv
    empty_f = jnp.sum(m_f, axis=1, keepdims=True) == 0.0
    empty_b = jnp.sum(m_b, axis=1, keepdims=True) == 0.0

    efs = []
    ebs = []
    big = jnp.float32(1e9)
    for h in range(NHEAD):
        raw_h = rawl[:, h * L:(h + 1) * L]
        mx = jnp.max(raw_h, axis=1, keepdims=True)
        e = jnp.exp(raw_h - mx)
        e2 = ((raw_h - big) >= (mx - big)).astype(jnp.float32)
        efs.append(jnp.where(empty_f, e2, e * m_f).astype(jnp.bfloat16))
        ebs.append(jnp.where(empty_b, e2, e * m_b).astype(jnp.bfloat16))
    ef = jnp.concatenate(efs, axis=1)
    eb = jnp.concatenate(ebs, axis=1)

    vt = jnp.concatenate([v, v, v, v], axis=0)
    vst = jnp.where(rh == ch, vt, 0.0)
    r4 = jax.lax.broadcasted_iota(jnp.int32, (NHEAD * L, NHEAD), 0) // L
    c4 = jax.lax.broadcasted_iota(jnp.int32, (NHEAD * L, NHEAD), 1)
    den_ind = (r4 == c4).astype(jnp.float32)
    vd = jnp.concatenate([vst, den_ind], axis=1).astype(jnp.bfloat16)

    ctf = jnp.dot(ef, vd, preferred_element_type=jnp.float32)
    ctb = jnp.dot(eb, vd, preferred_element_type=jnp.float32)
    outs = []
    for ct in (ctf, ctb):
        rec = 1.0 / ct[:, D:D + NHEAD]
        parts = [ct[:, h * DHEAD:(h + 1) * DHEAD] * rec[:, h:h + 1]
                 for h in range(NHEAD)]
        outs.append(jnp.concatenate(parts, axis=1))
    ctx2 = jnp.concatenate(outs, axis=0)
    attn2 = jnp.dot(ctx2, w_o, preferred_element_type=jnp.float32) + b_o

    def layer_norm(y, gg, bb):
        mu = jnp.mean(y, axis=-1, keepdims=True)
        m2 = jnp.mean(y * y, axis=-1, keepdims=True)
        var = m2 - mu * mu
        return (y - mu) * jax.lax.rsqrt(var + LN_EPS) * gg + bb

    x2 = jnp.concatenate([x, x], axis=0)
    x1 = layer_norm(x2 + attn2, ln1_g, ln1_b)
    hidden = jax.nn.relu(
        jnp.dot(x1, w_ff1, preferred_element_type=jnp.float32) + b_ff1)
    ff = jnp.dot(hidden, w_ff2, preferred_element_type=jnp.float32) + b_ff2
    y2 = layer_norm(x1 + ff, ln2_g, ln2_b)

    out_ref[i] = y2[0:L, :] + y2[L:2 * L, :]


def _graph_encoder(graph_in, aa_graph, aa_graph_length, w_slab, vec_slab,
                   *, graphs_per_block=2):
    B, L, D = graph_in.shape
    gb = graphs_per_block
    return pl.pallas_call(
        _encoder_kernel,
        out_shape=jax.ShapeDtypeStruct((B, L, D), jnp.float32),
        grid_spec=pltpu.PrefetchScalarGridSpec(
            num_scalar_prefetch=1,
            grid=(B // gb,),
            in_specs=[
                pl.BlockSpec((gb, L, D), lambda b, ln: (b, 0, 0)),
                pl.BlockSpec((gb, L, L), lambda b, ln: (b, 0, 0)),
                pl.BlockSpec(w_slab.shape, lambda b, ln: (0, 0)),
                pl.BlockSpec(vec_slab.shape, lambda b, ln: (0, 0)),
            ],
            out_specs=pl.BlockSpec((gb, L, D), lambda b, ln: (b, 0, 0)),
        ),
        compiler_params=pltpu.CompilerParams(dimension_semantics=("parallel",)),
    )(aa_graph_length.astype(jnp.int32), graph_in,
      aa_graph.astype(jnp.float32), w_slab, vec_slab)


def kernel(dense_w, dense_b, w_slab, vec_slab, as_features, bert_last_hidden,
           bert_pooler, aa_len, map_AA, map_AA_idx, map_AS, map_AS_idx,
           aa_graph_length, aa_graph):
    B, L, _ = aa_graph.shape

    rela_v_inner = _node_features(bert_last_hidden, aa_len, bert_pooler,
                                  dense_w, dense_b)

    graph_in = jnp.zeros((B, L, IN_DIM), jnp.float32)
    graph_in = graph_in.at[map_AA, map_AA_idx].set(rela_v_inner)
    graph_in = graph_in.at[map_AS, map_AS_idx].set(as_features)

    mutual = _graph_encoder(graph_in, aa_graph, aa_graph_length,
                            w_slab, vec_slab)

    AA_features = mutual[map_AS, map_AS_idx]
    need = (aa_graph_length[map_AS] > 1).astype(jnp.float32)[:, None]
    blended = AA_features * need + as_features * (1.0 - need)
    return blended + as_features


# speedup vs baseline: 1.9121x; 1.0171x over previous
import numpy as np
import jax
import jax.numpy as jnp
from jax.experimental import pallas as pl
from jax.experimental.pallas import tpu as pltpu

BERT_HIDDEN = 32
HIDDEN = 16
IN_DIM = BERT_HIDDEN + HIDDEN
NHEAD = 4
DHEAD = IN_DIM // NHEAD
DIM_FF = HIDDEN
LN_EPS = 1e-5


def _node_kernel(x_ref, len_ref, pool_ref, w_ref, b_ref, out_ref):
    TN, S1, _ = x_ref.shape
    S = S1 - 1
    lens = len_ref[...]
    pos = jax.lax.broadcasted_iota(jnp.int32, (TN, S1), 1)
    valid = ((pos >= 1) & (pos <= lens)).astype(jnp.float32)[:, :, None]
    xm_sum = jnp.sum(x_ref[...] * valid, axis=1)
    node = jnp.dot(xm_sum, w_ref[...],
                   preferred_element_type=jnp.float32) + jnp.float32(S) * b_ref[...]
    out_ref[...] = jnp.concatenate([node, pool_ref[...]], axis=-1)


def _node_features(bert_last_hidden, aa_len, pooler, w, b, *, tile_n=128):
    n, S1, Hb = bert_last_hidden.shape
    n_pad = ((n + tile_n - 1) // tile_n) * tile_n
    lens = aa_len.reshape(-1, 1).astype(jnp.int32)
    if n_pad != n:
        p = n_pad - n
        bert_last_hidden = jnp.pad(bert_last_hidden, ((0, p), (0, 0), (0, 0)))
        lens = jnp.pad(lens, ((0, p), (0, 0)))
        pooler = jnp.pad(pooler, ((0, p), (0, 0)))
    out = pl.pallas_call(
        _node_kernel,
        out_shape=jax.ShapeDtypeStruct((n_pad, HIDDEN + BERT_HIDDEN), jnp.float32),
        grid_spec=pltpu.PrefetchScalarGridSpec(
            num_scalar_prefetch=0,
            grid=(n_pad // tile_n,),
            in_specs=[
                pl.BlockSpec((tile_n, S1, Hb), lambda i: (i, 0, 0)),
                pl.BlockSpec((tile_n, 1), lambda i: (i, 0)),
                pl.BlockSpec((tile_n, Hb), lambda i: (i, 0)),
                pl.BlockSpec((Hb, HIDDEN), lambda i: (0, 0)),
                pl.BlockSpec((1, HIDDEN), lambda i: (0, 0)),
            ],
            out_specs=pl.BlockSpec((tile_n, HIDDEN + BERT_HIDDEN), lambda i: (i, 0)),
        ),
        compiler_params=pltpu.CompilerParams(dimension_semantics=("parallel",)),
    )(bert_last_hidden, lens, pooler, w, b)
    return out[:n]


def _encoder_kernel(len_ref, x_ref, g_ref, w_ref, vec_ref, out_ref):
    nb = x_ref.shape[0]
    for i in range(nb):
        _encode_one(len_ref, x_ref, g_ref, w_ref, vec_ref, out_ref,
                    pl.program_id(0) * nb + i, i)


def _encode_one(len_ref, x_ref, g_ref, w_ref, vec_ref, out_ref, b, i):
    x = x_ref[i]
    g = g_ref[i]
    L = x.shape[0]
    D = IN_DIM

    w_qkv = w_ref[0:D, :]
    w_o = w_ref[D:2 * D, 0:D]
    w_ff1 = w_ref[2 * D:3 * D, 0:DIM_FF]
    w_ff2 = w_ref[3 * D:3 * D + DIM_FF, 0:D]
    b_qkv = vec_ref[0:1, :]
    b_o = vec_ref[1:2, 0:D]
    b_ff1 = vec_ref[2:3, 0:DIM_FF]
    b_ff2 = vec_ref[3:4, 0:D]
    ln1_g = vec_ref[4:5, 0:D]
    ln1_b = vec_ref[5:6, 0:D]
    ln2_g = vec_ref[6:7, 0:D]
    ln2_b = vec_ref[7:8, 0:D]

    qkv = jnp.dot(x, w_qkv, preferred_element_type=jnp.float32) + b_qkv
    scale = jnp.float32(1.0 / np.sqrt(DHEAD))
    q = qkv[:, 0:D] * scale
    k = qkv[:, D:2 * D]
    v = qkv[:, 2 * D:3 * D]

    kt = jnp.concatenate([k, k, k, k], axis=0)
    rh = jax.lax.broadcasted_iota(jnp.int32, (NHEAD * L, D), 0) // L
    ch = jax.lax.broadcasted_iota(jnp.int32, (NHEAD * L, D), 1) // DHEAD
    kst = jnp.where(rh == ch, kt, 0.0)
    rawl = jax.lax.dot_general(q, kst, (((1,), (1,)), ((), ())),
                               preferred_element_type=jnp.float32)

    length = len_ref[b]
    col = jax.lax.broadcasted_iota(jnp.int32, (L, L), 1)
    cv = (col < length).astype(jnp.float32)
    a = (g != 0.0).astype(jnp.float32)
    m_f = a * cv
    m_b = a.T * cv
    empty_f = jnp.sum(m_f, axis=1, keepdims=True) == 0.0
    empty_b = jnp.sum(m_b, axis=1, keepdims=True) == 0.0

    efs = []
    ebs = []
    big = jnp.float32(1e9)
    for h in range(NHEAD):
        raw_h = rawl[:, h * L:(h + 1) * L]
        mx = jnp.max(raw_h, axis=1, keepdims=True)
        e = jnp.exp(raw_h - mx)
        e2 = ((raw_h - big) >= (mx - big)).astype(jnp.float32)
        efs.append(jnp.where(empty_f, e2, e * m_f).astype(jnp.bfloat16))
        ebs.append(jnp.where(empty_b, e2, e * m_b).astype(jnp.bfloat16))
    ef = jnp.concatenate(efs, axis=1)
    eb = jnp.concatenate(ebs, axis=1)

    vt = jnp.concatenate([v, v, v, v], axis=0)
    vst = jnp.where(rh == ch, vt, 0.0)
    r4 = jax.lax.broadcasted_iota(jnp.int32, (NHEAD * L, NHEAD), 0) // L
    c4 = jax.lax.broadcasted_iota(jnp.int32, (NHEAD * L, NHEAD), 1)
    den_ind = (r4 == c4).astype(jnp.float32)
    vd = jnp.concatenate([vst, den_ind], axis=1).astype(jnp.bfloat16)

    ctf = jnp.dot(ef, vd, preferred_element_type=jnp.float32)
    ctb = jnp.dot(eb, vd, preferred_element_type=jnp.float32)
    outs = []
    for ct in (ctf, ctb):
        rec = 1.0 / ct[:, D:D + NHEAD]
        parts = [ct[:, h * DHEAD:(h + 1) * DHEAD] * rec[:, h:h + 1]
                 for h in range(NHEAD)]
        outs.append(jnp.concatenate(parts, axis=1))
    ctx2 = jnp.concatenate(outs, axis=0)
    attn2 = jnp.dot(ctx2, w_o, preferred_element_type=jnp.float32) + b_o

    def layer_norm(y, gg, bb):
        mu = jnp.mean(y, axis=-1, keepdims=True)
        m2 = jnp.mean(y * y, axis=-1, keepdims=True)
        var = m2 - mu * mu
        return (y - mu) * jax.lax.rsqrt(var + LN_EPS) * gg + bb

    x2 = jnp.concatenate([x, x], axis=0)
    x1 = layer_norm(x2 + attn2, ln1_g, ln1_b)
    hidden = jax.nn.relu(
        jnp.dot(x1, w_ff1, preferred_element_type=jnp.float32) + b_ff1)
    ff = jnp.dot(hidden, w_ff2, preferred_element_type=jnp.float32) + b_ff2
    y2 = layer_norm(x1 + ff, ln2_g, ln2_b)

    out_ref[i] = y2[0:L, :] + y2[L:2 * L, :]


def _graph_encoder(graph_in, aa_graph, aa_graph_length, w_slab, vec_slab,
                   *, graphs_per_block=4):
    B, L, D = graph_in.shape
    gb = graphs_per_block
    return pl.pallas_call(
        _encoder_kernel,
        out_shape=jax.ShapeDtypeStruct((B, L, D), jnp.float32),
        grid_spec=pltpu.PrefetchScalarGridSpec(
            num_scalar_prefetch=1,
            grid=(B // gb,),
            in_specs=[
                pl.BlockSpec((gb, L, D), lambda b, ln: (b, 0, 0)),
                pl.BlockSpec((gb, L, L), lambda b, ln: (b, 0, 0)),
                pl.BlockSpec(w_slab.shape, lambda b, ln: (0, 0)),
                pl.BlockSpec(vec_slab.shape, lambda b, ln: (0, 0)),
            ],
            out_specs=pl.BlockSpec((gb, L, D), lambda b, ln: (b, 0, 0)),
        ),
        compiler_params=pltpu.CompilerParams(dimension_semantics=("parallel",)),
    )(aa_graph_length.astype(jnp.int32), graph_in,
      aa_graph.astype(jnp.float32), w_slab, vec_slab)


def kernel(dense_w, dense_b, w_slab, vec_slab, as_features, bert_last_hidden,
           bert_pooler, aa_len, map_AA, map_AA_idx, map_AS, map_AS_idx,
           aa_graph_length, aa_graph):
    B, L, _ = aa_graph.shape

    rela_v_inner = _node_features(bert_last_hidden, aa_len, bert_pooler,
                                  dense_w, dense_b)

    graph_in = jnp.zeros((B, L, IN_DIM), jnp.float32)
    graph_in = graph_in.at[map_AA, map_AA_idx].set(rela_v_inner)
    graph_in = graph_in.at[map_AS, map_AS_idx].set(as_features)

    mutual = _graph_encoder(graph_in, aa_graph, aa_graph_length,
                            w_slab, vec_slab)

    AA_features = mutual[map_AS, map_AS_idx]
    need = (aa_graph_length[map_AS] > 1).astype(jnp.float32)[:, None]
    blended = AA_features * need + as_features * (1.0 - need)
    return blended + as_features
